# Initial kernel scaffold; baseline (speedup 1.0000x reference)
import jax
import jax.numpy as jnp
from jax.experimental import pallas as pl


def kernel(x, idx):
    raise NotImplementedError("write your pallas kernel here")


# speedup vs baseline: 3.5210x; 3.5210x over previous
import functools

import jax
import jax.numpy as jnp
from jax import lax
from jax.experimental import pallas as pl
from jax.experimental.pallas import tpu as pltpu
from jax.experimental.pallas import tpu_sc as plsc

B, C, N_IN, N_OUT, K = 2, 256, 50000, 12500, 8
R = B * C
NP = 12512
G = NP // 32
NW = 32
ROWS_PER_W = R // NW


def _body(x_hbm, pidx_hbm, out_hbm, row_v, idx_v, out_v):
    wid = lax.axis_index("c") * 16 + lax.axis_index("s")

    pltpu.sync_copy(pidx_hbm, idx_v)

    def do_group(g, _):
        acc_a = None
        acc_b = None
        for k in range(K):
            p = idx_v[k, pl.ds(g * 16, 16)]
            a = p & 0xFFFF
            b = (p >> 16) & 0xFFFF
            va = plsc.load_gather(row_v, [a])
            vb = plsc.load_gather(row_v, [b])
            acc_a = va if acc_a is None else jnp.maximum(acc_a, va)
            acc_b = vb if acc_b is None else jnp.maximum(acc_b, vb)
        out_v[pl.ds(g * 32, 16)] = acc_a
        out_v[pl.ds(g * 32 + 16, 16)] = acc_b
        return _

    for r in range(ROWS_PER_W):
        row_id = wid * ROWS_PER_W + r
        pltpu.sync_copy(x_hbm.at[row_id], row_v)
        lax.fori_loop(0, G, do_group, 0)
        pltpu.sync_copy(out_v, out_hbm.at[row_id])


_sc_call = functools.partial(
    pl.kernel,
    out_type=jax.ShapeDtypeStruct((R, NP), jnp.float32),
    mesh=plsc.VectorSubcoreMesh(core_axis_name="c", subcore_axis_name="s"),
    compiler_params=pltpu.CompilerParams(needs_layout_passes=False),
    scratch_types=[
        pltpu.VMEM((N_IN,), jnp.float32),
        pltpu.VMEM((K, NP // 2), jnp.int32),
        pltpu.VMEM((NP,), jnp.float32),
    ],
)(_body)


def kernel(x, idx):
    xr = x.reshape(R, N_IN)
    idxp = jnp.concatenate([idx, jnp.zeros((NP - N_OUT, K), jnp.int32)], axis=0)
    t = idxp.T.reshape(K, G, 2, 16)
    packed = (t[:, :, 0, :] | (t[:, :, 1, :] << 16)).reshape(K, NP // 2)
    out = _sc_call(xr, packed)
    return out[:, :N_OUT].reshape(B, C, N_OUT)
